# baseline (speedup 1.0000x reference)
import functools

import jax
import jax.numpy as jnp
from jax import lax
from jax.experimental import pallas as pl
from jax.experimental.pallas import tpu as pltpu
from jax.experimental.pallas import tpu_sc as plsc

NC = 2
NS = 16
NW = NC * NS
C = 128
NBUF = 4
L = 16

D_MODEL = 64


def _make_gather(total, d):
    assert total % (NW * C) == 0
    b_per_w = total // NW
    nchunk = b_per_w // C
    assert nchunk % NBUF == 0
    mesh = plsc.VectorSubcoreMesh(core_axis_name="c", subcore_axis_name="s")

    @functools.partial(
        pl.kernel,
        mesh=mesh,
        compiler_params=pltpu.CompilerParams(
            use_tc_tiling_on_sc=False,
            disable_bounds_checks=True,
        ),
        out_type=jax.ShapeDtypeStruct((total, d), jnp.float32),
        scratch_types=[
            pltpu.VMEM((b_per_w,), jnp.int32),
            pltpu.VMEM((NBUF, C, d), jnp.float32),
            [pltpu.SemaphoreType.DMA] * NBUF,
        ],
    )
    def gather_kernel(idx_hbm, table_hbm, out_hbm, idx_v, rows_v, sems):
        cid = lax.axis_index("c")
        sid = lax.axis_index("s")
        wid = sid * NC + cid
        base = wid * b_per_w
        pltpu.sync_copy(idx_hbm.at[pl.ds(base, b_per_w)], idx_v)

        def fire(j, b):
            for k in range(C // L):
                iv = idx_v[pl.ds(j * C + k * L, L)]
                for i in range(L):
                    pltpu.async_copy(
                        table_hbm.at[pl.ds(iv[i], 1)],
                        rows_v.at[b, pl.ds(k * L + i, 1)],
                        sems[b],
                    )

        for b in range(NBUF):
            fire(b, b)

        def round_body(r, carry):
            j0 = r * NBUF
            for b in range(NBUF):
                j = j0 + b
                pltpu.make_async_copy(
                    table_hbm.at[idx_v[pl.ds(j * C, C)]], rows_v.at[b], sems[b]
                ).wait()
                pltpu.sync_copy(rows_v.at[b], out_hbm.at[pl.ds(base + j * C, C)])

                @pl.when(j + NBUF < nchunk)
                def _():
                    fire(j + NBUF, b)
            return carry

        lax.fori_loop(0, nchunk // NBUF, round_body, 0)

    return gather_kernel


def kernel(token_ids, weight):
    b, s = token_ids.shape
    d = weight.shape[1]
    total = b * s
    idx = token_ids.reshape(total).astype(jnp.int32)
    out = _make_gather(total, d)(idx, weight)
    return out.reshape(b, s, d)


# speedup vs baseline: 1.0419x; 1.0419x over previous
import functools

import jax
import jax.numpy as jnp
from jax import lax
from jax.experimental import pallas as pl
from jax.experimental.pallas import tpu as pltpu
from jax.experimental.pallas import tpu_sc as plsc

NC = 2
NS = 16
NW = NC * NS
C = 512
NBUF = 2

D_MODEL = 64


def _make_gather(total, d):
    assert total % (NW * C) == 0
    b_per_w = total // NW
    nchunk = b_per_w // C
    assert nchunk % NBUF == 0
    mesh = plsc.VectorSubcoreMesh(core_axis_name="c", subcore_axis_name="s")

    @functools.partial(
        pl.kernel,
        mesh=mesh,
        compiler_params=pltpu.CompilerParams(
            use_tc_tiling_on_sc=False,
            disable_bounds_checks=True,
        ),
        out_type=jax.ShapeDtypeStruct((total, d), jnp.float32),
        scratch_types=[
            pltpu.VMEM((b_per_w,), jnp.int32),
            pltpu.VMEM((NBUF, C, d), jnp.float32),
            pltpu.VMEM_SHARED((NS, C), jnp.int32),
            pltpu.SMEM((NBUF, C), jnp.int32),
            [pltpu.SemaphoreType.DMA] * NBUF,
        ],
    )
    def gather_kernel(idx_hbm, table_hbm, out_hbm, idx_v, rows_v, sp_i, sm_i, sems):
        cid = lax.axis_index("c")
        sid = lax.axis_index("s")
        wid = sid * NC + cid
        base = wid * b_per_w
        pltpu.sync_copy(idx_hbm.at[pl.ds(base, b_per_w)], idx_v)

        def stage(j, b):
            pltpu.sync_copy(idx_v.at[pl.ds(j * C, C)], sp_i.at[sid])
            pltpu.sync_copy(sp_i.at[sid], sm_i.at[b])

        def fire(b):
            def dma_one(i, carry):
                s = sm_i[b, i]
                pltpu.async_copy(
                    table_hbm.at[pl.ds(s, 1)], rows_v.at[b, pl.ds(i, 1)], sems[b]
                )
                return carry

            lax.fori_loop(0, C, dma_one, 0, unroll=8)

        for b in range(NBUF):
            stage(b, b)
            fire(b)

        def round_body(r, carry):
            j0 = r * NBUF
            for b in range(NBUF):
                j = j0 + b
                pltpu.make_async_copy(
                    table_hbm.at[pl.ds(0, C)], rows_v.at[b], sems[b]
                ).wait()
                pltpu.sync_copy(rows_v.at[b], out_hbm.at[pl.ds(base + j * C, C)])

                @pl.when(j + NBUF < nchunk)
                def _():
                    stage(j + NBUF, b)
                    fire(b)
            return carry

        lax.fori_loop(0, nchunk // NBUF, round_body, 0)

    return gather_kernel


def kernel(token_ids, weight):
    b, s = token_ids.shape
    d = weight.shape[1]
    total = b * s
    idx = token_ids.reshape(total).astype(jnp.int32)
    out = _make_gather(total, d)(idx, weight)
    return out.reshape(b, s, d)
